# Initial kernel scaffold; baseline (speedup 1.0000x reference)
import jax
import jax.numpy as jnp
from jax.experimental import pallas as pl


def kernel(token_ids, embedding_matrix):
    raise NotImplementedError("write your pallas kernel here")


# speedup vs baseline: 1.1139x; 1.1139x over previous
import functools

import jax
import jax.numpy as jnp
from jax import lax
from jax.experimental import pallas as pl
from jax.experimental.pallas import tpu as pltpu
from jax.experimental.pallas import tpu_sc as plsc

NUM_TABLE_ROWS = 1000000
DIM = 32
CHUNK = 128
CPG = 10
GROUP_ROWS = CHUNK * CPG


@functools.partial(jax.jit, static_argnums=(2, 3))
def _sc_gather(idx3, table, n_workers, n_chunks_pw):
    n_rows = n_workers * n_chunks_pw * CHUNK
    rows_pw = n_chunks_pw * CHUNK
    n_groups = n_chunks_pw // CPG

    mesh = plsc.VectorSubcoreMesh(core_axis_name="c", subcore_axis_name="s")

    @functools.partial(
        pl.kernel,
        mesh=mesh,
        out_type=jax.ShapeDtypeStruct((n_rows, DIM), jnp.float32),
        scratch_types=[
            pltpu.VMEM((n_chunks_pw, CHUNK), jnp.int32),
            pltpu.VMEM((2, GROUP_ROWS, DIM), jnp.float32),
            pltpu.SemaphoreType.DMA((2,)),
        ],
        compiler_params=pltpu.CompilerParams(use_tc_tiling_on_sc=False),
    )
    def k(idx_hbm, table_hbm, out_hbm, idx_v, rows_v, sems):
        n_cores = 2
        wid = lax.axis_index("s") * n_cores + lax.axis_index("c")
        base = wid * rows_pw
        pltpu.sync_copy(idx_hbm.at[wid], idx_v)

        def start_group(g, buf):
            for j in range(CPG):
                c = g * CPG + j
                pltpu.make_async_copy(
                    table_hbm.at[idx_v.at[c]],
                    rows_v.at[buf].at[pl.ds(j * CHUNK, CHUNK)],
                    sems.at[buf],
                ).start()

        def wait_group(buf):
            for j in range(CPG):
                pltpu.make_async_copy(
                    table_hbm.at[idx_v.at[0]],
                    rows_v.at[buf].at[pl.ds(j * CHUNK, CHUNK)],
                    sems.at[buf],
                ).wait()

        start_group(0, 0)

        def body(gp, _):
            for buf in range(2):
                g = gp * 2 + buf
                nxt = g + 1

                @pl.when(nxt < n_groups)
                def _():
                    start_group(nxt, 1 - buf)

                wait_group(buf)
                pltpu.sync_copy(
                    rows_v.at[buf],
                    out_hbm.at[pl.ds(base + g * GROUP_ROWS, GROUP_ROWS)],
                )
            return 0

        lax.fori_loop(0, n_groups // 2, body, 0, unroll=False)

    return k(idx3, table)


def kernel(token_ids, embedding_matrix):
    b, s = token_ids.shape
    n = b * s
    n_workers = 32
    rows_pw = n // n_workers
    n_chunks_pw = rows_pw // CHUNK
    idx3 = token_ids.reshape(n_workers, n_chunks_pw, CHUNK).astype(jnp.int32)
    out = _sc_gather(idx3, embedding_matrix, n_workers, n_chunks_pw)
    return out.reshape(b, s, DIM)


# speedup vs baseline: 1.1141x; 1.0001x over previous
import functools

import jax
import jax.numpy as jnp
from jax import lax
from jax.experimental import pallas as pl
from jax.experimental.pallas import tpu as pltpu
from jax.experimental.pallas import tpu_sc as plsc

NUM_TABLE_ROWS = 1000000
DIM = 32
CHUNK = 1280
CPG = 1
GROUP_ROWS = CHUNK * CPG


@functools.partial(jax.jit, static_argnums=(2, 3))
def _sc_gather(idx3, table, n_workers, n_chunks_pw):
    n_rows = n_workers * n_chunks_pw * CHUNK
    rows_pw = n_chunks_pw * CHUNK
    n_groups = n_chunks_pw // CPG

    mesh = plsc.VectorSubcoreMesh(core_axis_name="c", subcore_axis_name="s")

    @functools.partial(
        pl.kernel,
        mesh=mesh,
        out_type=jax.ShapeDtypeStruct((n_rows, DIM), jnp.float32),
        scratch_types=[
            pltpu.VMEM((n_chunks_pw, CHUNK), jnp.int32),
            pltpu.VMEM((2, GROUP_ROWS, DIM), jnp.float32),
            pltpu.SemaphoreType.DMA((2,)),
        ],
        compiler_params=pltpu.CompilerParams(use_tc_tiling_on_sc=False),
    )
    def k(idx_hbm, table_hbm, out_hbm, idx_v, rows_v, sems):
        n_cores = 2
        wid = lax.axis_index("s") * n_cores + lax.axis_index("c")
        base = wid * rows_pw
        pltpu.sync_copy(idx_hbm.at[wid], idx_v)

        def start_group(g, buf):
            for j in range(CPG):
                c = g * CPG + j
                pltpu.make_async_copy(
                    table_hbm.at[idx_v.at[c]],
                    rows_v.at[buf].at[pl.ds(j * CHUNK, CHUNK)],
                    sems.at[buf],
                ).start()

        def wait_group(buf):
            for j in range(CPG):
                pltpu.make_async_copy(
                    table_hbm.at[idx_v.at[0]],
                    rows_v.at[buf].at[pl.ds(j * CHUNK, CHUNK)],
                    sems.at[buf],
                ).wait()

        start_group(0, 0)

        def body(gp, _):
            for buf in range(2):
                g = gp * 2 + buf
                nxt = g + 1

                @pl.when(nxt < n_groups)
                def _():
                    start_group(nxt, 1 - buf)

                wait_group(buf)
                pltpu.sync_copy(
                    rows_v.at[buf],
                    out_hbm.at[pl.ds(base + g * GROUP_ROWS, GROUP_ROWS)],
                )
            return 0

        lax.fori_loop(0, n_groups // 2, body, 0, unroll=False)

    return k(idx3, table)


def kernel(token_ids, embedding_matrix):
    b, s = token_ids.shape
    n = b * s
    n_workers = 32
    rows_pw = n // n_workers
    n_chunks_pw = rows_pw // CHUNK
    idx3 = token_ids.reshape(n_workers, n_chunks_pw, CHUNK).astype(jnp.int32)
    out = _sc_gather(idx3, embedding_matrix, n_workers, n_chunks_pw)
    return out.reshape(b, s, DIM)


# speedup vs baseline: 1.8042x; 1.6195x over previous
import functools

import jax
import jax.numpy as jnp
from jax import lax
from jax.experimental import pallas as pl
from jax.experimental.pallas import tpu as pltpu
from jax.experimental.pallas import tpu_sc as plsc

DIM = 32
GROUP_B = 16


@functools.partial(jax.jit, static_argnums=(2,))
def _sc_gather(idx3, table, n_workers):
    _, b_pw, seq = idx3.shape
    n_groups = b_pw // GROUP_B

    mesh = plsc.VectorSubcoreMesh(core_axis_name="c", subcore_axis_name="s")

    @functools.partial(
        pl.kernel,
        mesh=mesh,
        out_type=jax.ShapeDtypeStruct((n_workers * b_pw, seq, DIM), jnp.float32),
        scratch_types=[
            pltpu.VMEM((b_pw, seq), jnp.int32),
            pltpu.VMEM((2, GROUP_B, seq, DIM), jnp.float32),
            pltpu.SemaphoreType.DMA((2,)),
        ],
        compiler_params=pltpu.CompilerParams(use_tc_tiling_on_sc=False),
    )
    def k(idx_hbm, table_hbm, out_hbm, idx_v, rows_v, sems):
        n_cores = 2
        wid = lax.axis_index("s") * n_cores + lax.axis_index("c")
        base = wid * b_pw
        pltpu.sync_copy(idx_hbm.at[wid], idx_v)

        def start_group(g, buf):
            for i in range(GROUP_B):
                pltpu.make_async_copy(
                    table_hbm.at[idx_v.at[g * GROUP_B + i]],
                    rows_v.at[buf, i],
                    sems.at[buf],
                ).start()

        def wait_group(buf):
            for i in range(GROUP_B):
                pltpu.make_async_copy(
                    table_hbm.at[idx_v.at[0]],
                    rows_v.at[buf, i],
                    sems.at[buf],
                ).wait()

        start_group(0, 0)

        def body(gp, _):
            for buf in range(2):
                g = gp * 2 + buf
                nxt = g + 1

                @pl.when(nxt < n_groups)
                def _():
                    start_group(nxt, 1 - buf)

                wait_group(buf)
                pltpu.sync_copy(
                    rows_v.at[buf],
                    out_hbm.at[pl.ds(base + g * GROUP_B, GROUP_B)],
                )
            return 0

        lax.fori_loop(0, n_groups // 2, body, 0, unroll=False)

    return k(idx3, table)


def kernel(token_ids, embedding_matrix):
    b, s = token_ids.shape
    n_workers = 32
    b_pw = b // n_workers
    idx3 = token_ids.reshape(n_workers, b_pw, s).astype(jnp.int32)
    return _sc_gather(idx3, embedding_matrix, n_workers)
